# baseline (speedup 1.0000x reference)
import jax
import jax.numpy as jnp
from jax import lax
from jax.experimental import pallas as pl
from jax.experimental.pallas import tpu as pltpu
from jax.experimental.pallas import tpu_sc as plsc

N = 1000
B = 16384
NC = 2
NS = 16
NW = NC * NS
BPW = B // NW
CHUNK = 128
NCH = BPW // CHUNK
NV = BPW // 16
RPAD = 1024
RB = 40
GSTEPS = N // RB
LBINS = GSTEPS * 128


def _tc_body(inp_ref, cab_ref, cba_ref,
             a_ref, b_ref, lA_ref, lB_ref):
    i = pl.program_id(0)

    @pl.when(i == 0)
    def _extract_ab():
        a_ref[:] = inp_ref[:, 0]
        b_ref[:] = inp_ref[:, 1]

    zpad = jnp.zeros((128 - RB,), jnp.float32)

    def do_table(c_ref, l_ref):
        blk = c_ref[:]
        m = jnp.max(blk, axis=1)
        lse = jnp.log(jnp.sum(jnp.exp(blk - m[:, None]), axis=1)) + m
        l_ref[:] = jnp.concatenate([lse, zpad])

    do_table(cab_ref, lA_ref)
    do_table(cba_ref, lB_ref)


_tc_call = pl.pallas_call(
    _tc_body,
    grid=(GSTEPS,),
    in_specs=[
        pl.BlockSpec((B, 2), lambda i: (0, 0)),
        pl.BlockSpec((RB, N), lambda i: (i, 0)),
        pl.BlockSpec((RB, N), lambda i: (i, 0)),
    ],
    out_specs=[
        pl.BlockSpec((B,), lambda i: (0,)),
        pl.BlockSpec((B,), lambda i: (0,)),
        pl.BlockSpec((128,), lambda i: (i,)),
        pl.BlockSpec((128,), lambda i: (i,)),
    ],
    out_shape=(
        jax.ShapeDtypeStruct((B,), jnp.int32),
        jax.ShapeDtypeStruct((B,), jnp.int32),
        jax.ShapeDtypeStruct((LBINS,), jnp.float32),
        jax.ShapeDtypeStruct((LBINS,), jnp.float32),
    ),
)


def _sc_body(a_hbm, b_hbm, wmA_hbm, wmB_hbm, fA_hbm, fB_hbm,
             outA_hbm, outB_hbm, cntA_hbm, cntB_hbm,
             a_v, b_v, idxA, idxB, idxTA, idxTB, idxHA, idxHB, gA, gB, gmA, gmB,
             ones_v, zeros_v, accA_v, accB_v, hist_a, hist_b, sem):
    cid = lax.axis_index("c")
    sid = lax.axis_index("s")
    wid = sid * NC + cid
    base = wid * BPW
    pltpu.sync_copy(a_hbm.at[pl.ds(base, BPW)], a_v)
    pltpu.sync_copy(b_hbm.at[pl.ds(base, BPW)], b_v)
    for k in range(8):
        ones_v[pl.ds(16 * k, 16)] = jnp.ones((16,), jnp.float32)

    @pl.when(sid == 0)
    def _zero_hist():
        for k in range(LBINS // 16):
            zeros_v[pl.ds(16 * k, 16)] = jnp.zeros((16,), jnp.float32)
        pltpu.sync_copy(zeros_v, hist_a)
        pltpu.sync_copy(zeros_v, hist_b)

    pair_cp = []
    for j in range(NV):
        a16 = a_v[pl.ds(16 * j, 16)]
        b16 = b_v[pl.ds(16 * j, 16)]
        r, s = j // 8, pl.ds(16 * (j % 8), 16)
        idxA[r, s] = a16 * N + b16
        idxB[r, s] = b16 * N + a16
        idxTA[r, s] = a16
        idxTB[r, s] = b16
    for c in range(NCH):
        pair_cp.append(pltpu.async_copy(fA_hbm.at[idxA.at[c]], gA.at[c], sem))
        pair_cp.append(pltpu.async_copy(fB_hbm.at[idxB.at[c]], gB.at[c], sem))
        pair_cp.append(pltpu.async_copy(wmA_hbm.at[idxTA.at[c]], gmA.at[c], sem))
        pair_cp.append(pltpu.async_copy(wmB_hbm.at[idxTB.at[c]], gmB.at[c], sem))

    for j in range(NV):
        r, s = j // 8, pl.ds(16 * (j % 8), 16)
        a16 = idxTA[r, s]
        b16 = idxTB[r, s]
        qa = (a16 * 52429) >> 21
        qb = (b16 * 52429) >> 21
        idxHA[r, s] = (qa << 7) + a16 - RB * qa
        idxHB[r, s] = (qb << 7) + b16 - RB * qb

    plsc.subcore_barrier()
    for c in range(NCH):
        pltpu.sync_copy(ones_v, hist_a.at[idxHA.at[c]], add=True)
        pltpu.sync_copy(ones_v, hist_b.at[idxHB.at[c]], add=True)
    plsc.subcore_barrier()

    @pl.when(sid == 0)
    def _write_hist():
        pltpu.sync_copy(hist_a, cntA_hbm.at[cid])
        pltpu.sync_copy(hist_b, cntB_hbm.at[cid])

    for cp in pair_cp:
        cp.wait()
    accA = jnp.zeros((16,), jnp.float32)
    accB = jnp.zeros((16,), jnp.float32)
    for j in range(NV):
        r, s = j // 8, pl.ds(16 * (j % 8), 16)
        accA = accA + gA[r, s] + gmA[r, s]
        accB = accB + gB[r, s] + gmB[r, s]
    accA_v[:] = accA
    accB_v[:] = accB
    pltpu.sync_copy(accA_v, outA_hbm.at[wid])
    pltpu.sync_copy(accB_v, outB_hbm.at[wid])


_sc_call = pl.kernel(
    _sc_body,
    out_type=(
        jax.ShapeDtypeStruct((NW, 16), jnp.float32),
        jax.ShapeDtypeStruct((NW, 16), jnp.float32),
        jax.ShapeDtypeStruct((NC, LBINS), jnp.float32),
        jax.ShapeDtypeStruct((NC, LBINS), jnp.float32),
    ),
    mesh=plsc.VectorSubcoreMesh(core_axis_name="c", subcore_axis_name="s"),
    scratch_types=(
        pltpu.VMEM((BPW,), jnp.int32),
        pltpu.VMEM((BPW,), jnp.int32),
        pltpu.VMEM((NCH, CHUNK), jnp.int32),
        pltpu.VMEM((NCH, CHUNK), jnp.int32),
        pltpu.VMEM((NCH, CHUNK), jnp.int32),
        pltpu.VMEM((NCH, CHUNK), jnp.int32),
        pltpu.VMEM((NCH, CHUNK), jnp.int32),
        pltpu.VMEM((NCH, CHUNK), jnp.int32),
        pltpu.VMEM((NCH, CHUNK), jnp.float32),
        pltpu.VMEM((NCH, CHUNK), jnp.float32),
        pltpu.VMEM((NCH, CHUNK), jnp.float32),
        pltpu.VMEM((NCH, CHUNK), jnp.float32),
        pltpu.VMEM((CHUNK,), jnp.float32),
        pltpu.VMEM((LBINS,), jnp.float32),
        pltpu.VMEM((16,), jnp.float32),
        pltpu.VMEM((16,), jnp.float32),
        pltpu.VMEM_SHARED((LBINS,), jnp.float32),
        pltpu.VMEM_SHARED((LBINS,), jnp.float32),
        pltpu.SemaphoreType.DMA,
    ),
)


def _combine_body(w_ref, wmA_ref, wmB_ref, lA_ref, lB_ref,
                  cntA_ref, cntB_ref, pA_ref, pB_ref, out_ref):
    def lse1d(v):
        m = jnp.max(v)
        return jnp.log(jnp.sum(jnp.exp(v - m))) + m

    cA = cntA_ref[0, :] + cntA_ref[1, :]
    cB = cntB_ref[0, :] + cntB_ref[1, :]
    S_AB = jnp.sum(pA_ref[:]) - B * lse1d(wmA_ref[:]) - jnp.sum(cA * lA_ref[:])
    S_BA = jnp.sum(pB_ref[:]) - B * lse1d(wmB_ref[:]) - jnp.sum(cB * lB_ref[:])
    wv = w_ref[:, :]
    la = -jnp.log(1.0 + jnp.exp(-wv))
    l1a = -jnp.log(1.0 + jnp.exp(wv))
    x = la + S_AB
    y = l1a + S_BA
    m = jnp.maximum(x, y)
    out_ref[:, :] = m + jnp.log(jnp.exp(x - m) + jnp.exp(y - m))


_combine_call = pl.pallas_call(
    _combine_body,
    out_shape=jax.ShapeDtypeStruct((1, 1), jnp.float32),
)


def kernel(inputs, w, w_mA, w_cAB, w_mB, w_cBA):
    a, b, lA, lB = _tc_call(inputs, w_cAB, w_cBA)
    outA, outB, cntA, cntB = _sc_call(a, b, w_mA, w_mB,
                                      w_cAB.reshape(-1), w_cBA.reshape(-1))
    res = _combine_call(jnp.reshape(w, (1, 1)), w_mA, w_mB, lA, lB,
                        cntA, cntB, outA, outB)
    return jnp.reshape(res, ())


# speedup vs baseline: 1.5307x; 1.5307x over previous
import jax
import jax.numpy as jnp
from jax import lax
from jax.experimental import pallas as pl
from jax.experimental.pallas import tpu as pltpu
from jax.experimental.pallas import tpu_sc as plsc

N = 1000
B = 16384
NC = 2
NS = 16
NW = NC * NS
BPW = B // NW
CHUNK = 128
NCH = BPW // CHUNK
NV = BPW // 16


def _tc_body(wmA_ref, cab_ref, wmB_ref, cba_ref, tA_ref, tB_ref):
    def t_for(wm, c):
        m = jnp.max(c, axis=1)
        lse = jnp.log(jnp.sum(jnp.exp(c - m[:, None]), axis=1)) + m
        mm = jnp.max(wm)
        lse_m = jnp.log(jnp.sum(jnp.exp(wm - mm))) + mm
        return wm - lse_m - lse

    tA_ref[:] = t_for(wmA_ref[:], cab_ref[:])
    tB_ref[:] = t_for(wmB_ref[:], cba_ref[:])


_tc_call = pl.pallas_call(
    _tc_body,
    out_shape=(
        jax.ShapeDtypeStruct((N,), jnp.float32),
        jax.ShapeDtypeStruct((N,), jnp.float32),
    ),
)


def _sc_body(a_hbm, b_hbm, tA_hbm, tB_hbm, wab_hbm, wba_hbm,
             outA_hbm, outB_hbm,
             a_v, b_v, idxA, idxB, idxTA, idxTB, gA, gB, gtA, gtB,
             accA_v, accB_v, sem):
    wid = lax.axis_index("s") * NC + lax.axis_index("c")
    base = wid * BPW
    pltpu.sync_copy(a_hbm.at[pl.ds(base, BPW)], a_v)
    pltpu.sync_copy(b_hbm.at[pl.ds(base, BPW)], b_v)
    for j in range(NV):
        a16 = a_v[pl.ds(16 * j, 16)]
        b16 = b_v[pl.ds(16 * j, 16)]
        idxA[j // 8, pl.ds(16 * (j % 8), 16)] = a16 * N + b16
        idxB[j // 8, pl.ds(16 * (j % 8), 16)] = b16 * N + a16
        idxTA[j // 8, pl.ds(16 * (j % 8), 16)] = a16
        idxTB[j // 8, pl.ds(16 * (j % 8), 16)] = b16
    copies = []
    for c in range(NCH):
        copies.append(pltpu.async_copy(wab_hbm.at[idxA.at[c]], gA.at[c], sem))
        copies.append(pltpu.async_copy(wba_hbm.at[idxB.at[c]], gB.at[c], sem))
        copies.append(pltpu.async_copy(tA_hbm.at[idxTA.at[c]], gtA.at[c], sem))
        copies.append(pltpu.async_copy(tB_hbm.at[idxTB.at[c]], gtB.at[c], sem))
    for cp in copies:
        cp.wait()
    accA = jnp.zeros((16,), jnp.float32)
    accB = jnp.zeros((16,), jnp.float32)
    for j in range(NV):
        r, s = j // 8, pl.ds(16 * (j % 8), 16)
        accA = accA + gA[r, s] + gtA[r, s]
        accB = accB + gB[r, s] + gtB[r, s]
    accA_v[:] = accA
    accB_v[:] = accB
    pltpu.sync_copy(accA_v, outA_hbm.at[wid])
    pltpu.sync_copy(accB_v, outB_hbm.at[wid])


_sc_call = pl.kernel(
    _sc_body,
    out_type=(
        jax.ShapeDtypeStruct((NW, 16), jnp.float32),
        jax.ShapeDtypeStruct((NW, 16), jnp.float32),
    ),
    mesh=plsc.VectorSubcoreMesh(core_axis_name="c", subcore_axis_name="s"),
    scratch_types=(
        pltpu.VMEM((BPW,), jnp.int32),
        pltpu.VMEM((BPW,), jnp.int32),
        pltpu.VMEM((NCH, CHUNK), jnp.int32),
        pltpu.VMEM((NCH, CHUNK), jnp.int32),
        pltpu.VMEM((NCH, CHUNK), jnp.int32),
        pltpu.VMEM((NCH, CHUNK), jnp.int32),
        pltpu.VMEM((NCH, CHUNK), jnp.float32),
        pltpu.VMEM((NCH, CHUNK), jnp.float32),
        pltpu.VMEM((NCH, CHUNK), jnp.float32),
        pltpu.VMEM((NCH, CHUNK), jnp.float32),
        pltpu.VMEM((16,), jnp.float32),
        pltpu.VMEM((16,), jnp.float32),
        pltpu.SemaphoreType.DMA,
    ),
)


def kernel(inputs, w, w_mA, w_cAB, w_mB, w_cBA):
    a = inputs[:, 0]
    b = inputs[:, 1]
    tA, tB = _tc_call(w_mA, w_cAB, w_mB, w_cBA)
    outA, outB = _sc_call(a, b, tA, tB, w_cAB.reshape(-1), w_cBA.reshape(-1))
    S_AB = jnp.sum(outA)
    S_BA = jnp.sum(outB)
    return jnp.logaddexp(jax.nn.log_sigmoid(w) + S_AB,
                         jax.nn.log_sigmoid(-w) + S_BA)


# speedup vs baseline: 1.6151x; 1.0551x over previous
import jax
import jax.numpy as jnp
from jax import lax
from jax.experimental import pallas as pl
from jax.experimental.pallas import tpu as pltpu
from jax.experimental.pallas import tpu_sc as plsc

N = 1000
B = 16384
NC = 2
NS = 16
NW = NC * NS
BPW = B // NW
CHUNK = 128
NCH = BPW // CHUNK
NV = BPW // 16
CB = 2048
CSTEPS = B // CB


def _tc_body(wmA_ref, cab_ref, wmB_ref, cba_ref, tA_ref, tB_ref):
    def t_for(wm, c):
        m = jnp.max(c, axis=1)
        lse = jnp.log(jnp.sum(jnp.exp(c - m[:, None]), axis=1)) + m
        mm = jnp.max(wm)
        lse_m = jnp.log(jnp.sum(jnp.exp(wm - mm))) + mm
        return wm - lse_m - lse

    tA_ref[:] = t_for(wmA_ref[:], cab_ref[:])
    tB_ref[:] = t_for(wmB_ref[:], cba_ref[:])


_tc_call = pl.pallas_call(
    _tc_body,
    out_shape=(
        jax.ShapeDtypeStruct((N,), jnp.float32),
        jax.ShapeDtypeStruct((N,), jnp.float32),
    ),
)


def _sc_body(a_hbm, b_hbm, wab_hbm, wba_hbm,
             outA_hbm, outB_hbm,
             a_v, b_v, idxA, idxB, gA, gB,
             accA_v, accB_v, sem):
    wid = lax.axis_index("s") * NC + lax.axis_index("c")
    base = wid * BPW
    pltpu.sync_copy(a_hbm.at[pl.ds(base, BPW)], a_v)
    pltpu.sync_copy(b_hbm.at[pl.ds(base, BPW)], b_v)
    for j in range(NV):
        a16 = a_v[pl.ds(16 * j, 16)]
        b16 = b_v[pl.ds(16 * j, 16)]
        idxA[j // 8, pl.ds(16 * (j % 8), 16)] = a16 * N + b16
        idxB[j // 8, pl.ds(16 * (j % 8), 16)] = b16 * N + a16
    copies = []
    for c in range(NCH):
        copies.append(pltpu.async_copy(wab_hbm.at[idxA.at[c]], gA.at[c], sem))
        copies.append(pltpu.async_copy(wba_hbm.at[idxB.at[c]], gB.at[c], sem))
    for cp in copies:
        cp.wait()
    accA = jnp.zeros((16,), jnp.float32)
    accB = jnp.zeros((16,), jnp.float32)
    for j in range(NV):
        r, s = j // 8, pl.ds(16 * (j % 8), 16)
        accA = accA + gA[r, s]
        accB = accB + gB[r, s]
    accA_v[:] = accA
    accB_v[:] = accB
    pltpu.sync_copy(accA_v, outA_hbm.at[wid])
    pltpu.sync_copy(accB_v, outB_hbm.at[wid])


_sc_call = pl.kernel(
    _sc_body,
    out_type=(
        jax.ShapeDtypeStruct((NW, 16), jnp.float32),
        jax.ShapeDtypeStruct((NW, 16), jnp.float32),
    ),
    mesh=plsc.VectorSubcoreMesh(core_axis_name="c", subcore_axis_name="s"),
    scratch_types=(
        pltpu.VMEM((BPW,), jnp.int32),
        pltpu.VMEM((BPW,), jnp.int32),
        pltpu.VMEM((NCH, CHUNK), jnp.int32),
        pltpu.VMEM((NCH, CHUNK), jnp.int32),
        pltpu.VMEM((NCH, CHUNK), jnp.float32),
        pltpu.VMEM((NCH, CHUNK), jnp.float32),
        pltpu.VMEM((16,), jnp.float32),
        pltpu.VMEM((16,), jnp.float32),
        pltpu.SemaphoreType.DMA,
    ),
)


def _combine_body(w_ref, a_ref, b_ref, tA_ref, tB_ref, pA_ref, pB_ref,
                  out_ref, cntA_acc, cntB_acc):
    i = pl.program_id(0)

    @pl.when(i == 0)
    def _init():
        cntA_acc[:] = jnp.zeros((N,), jnp.float32)
        cntB_acc[:] = jnp.zeros((N,), jnp.float32)

    cat = lax.broadcasted_iota(jnp.int32, (CB, N), 1)
    onehotA = (a_ref[:][:, None] == cat).astype(jnp.float32)
    onehotB = (b_ref[:][:, None] == cat).astype(jnp.float32)
    cntA_acc[:] += jnp.sum(onehotA, axis=0)
    cntB_acc[:] += jnp.sum(onehotB, axis=0)

    @pl.when(i == CSTEPS - 1)
    def _final():
        S_AB = jnp.sum(cntA_acc[:] * tA_ref[:]) + jnp.sum(pA_ref[:])
        S_BA = jnp.sum(cntB_acc[:] * tB_ref[:]) + jnp.sum(pB_ref[:])
        wv = w_ref[:, :]
        la = -jnp.log(1.0 + jnp.exp(-wv))
        l1a = -jnp.log(1.0 + jnp.exp(wv))
        x = la + S_AB
        y = l1a + S_BA
        m = jnp.maximum(x, y)
        out_ref[:, :] = m + jnp.log(jnp.exp(x - m) + jnp.exp(y - m))


_combine_call = pl.pallas_call(
    _combine_body,
    grid=(CSTEPS,),
    in_specs=[
        pl.BlockSpec((1, 1), lambda i: (0, 0)),
        pl.BlockSpec((CB,), lambda i: (i,)),
        pl.BlockSpec((CB,), lambda i: (i,)),
        pl.BlockSpec((N,), lambda i: (0,)),
        pl.BlockSpec((N,), lambda i: (0,)),
        pl.BlockSpec((NW, 16), lambda i: (0, 0)),
        pl.BlockSpec((NW, 16), lambda i: (0, 0)),
    ],
    out_specs=pl.BlockSpec((1, 1), lambda i: (0, 0)),
    out_shape=jax.ShapeDtypeStruct((1, 1), jnp.float32),
    scratch_shapes=[
        pltpu.VMEM((N,), jnp.float32),
        pltpu.VMEM((N,), jnp.float32),
    ],
)


def kernel(inputs, w, w_mA, w_cAB, w_mB, w_cBA):
    a = inputs[:, 0]
    b = inputs[:, 1]
    tA, tB = _tc_call(w_mA, w_cAB, w_mB, w_cBA)
    outA, outB = _sc_call(a, b, w_cAB.reshape(-1), w_cBA.reshape(-1))
    res = _combine_call(jnp.reshape(w, (1, 1)), a, b, tA, tB, outA, outB)
    return jnp.reshape(res, ())


# speedup vs baseline: 2.1872x; 1.3543x over previous
import jax
import jax.numpy as jnp
from jax import lax
from jax.experimental import pallas as pl
from jax.experimental.pallas import tpu as pltpu
from jax.experimental.pallas import tpu_sc as plsc

N = 1000
B = 16384
NC = 2
NS = 16
NW = NC * NS
BPW = B // NW
CHUNK = 128
NCH = BPW // CHUNK
NV = BPW // 16
CB = 2048
CSTEPS = B // CB


def _tc_body(wmA_ref, cab_ref, wmB_ref, cba_ref, tA_ref, tB_ref):
    def t_for(wm, c):
        m = jnp.max(c, axis=1)
        lse = jnp.log(jnp.sum(jnp.exp(c - m[:, None]), axis=1)) + m
        mm = jnp.max(wm)
        lse_m = jnp.log(jnp.sum(jnp.exp(wm - mm))) + mm
        return wm - lse_m - lse

    tA_ref[:] = t_for(wmA_ref[:], cab_ref[:])
    tB_ref[:] = t_for(wmB_ref[:], cba_ref[:])


_tc_call = pl.pallas_call(
    _tc_body,
    out_shape=(
        jax.ShapeDtypeStruct((N,), jnp.float32),
        jax.ShapeDtypeStruct((N,), jnp.float32),
    ),
)


def _sc_body(a_hbm, b_hbm, wab_hbm, wba_hbm,
             outA_hbm, outB_hbm,
             a_v, b_v, idxA, idxB, gA, gB,
             accA_v, accB_v, sem):
    wid = lax.axis_index("s") * NC + lax.axis_index("c")
    base = wid * BPW
    pltpu.sync_copy(a_hbm.at[pl.ds(base, BPW)], a_v)
    pltpu.sync_copy(b_hbm.at[pl.ds(base, BPW)], b_v)
    for j in range(NV):
        a16 = a_v[pl.ds(16 * j, 16)]
        b16 = b_v[pl.ds(16 * j, 16)]
        idxA[j // 8, pl.ds(16 * (j % 8), 16)] = a16 * N + b16
        idxB[j // 8, pl.ds(16 * (j % 8), 16)] = b16 * N + a16
    copies = []
    for c in range(NCH):
        copies.append(pltpu.async_copy(wab_hbm.at[idxA.at[c]], gA.at[c], sem))
        copies.append(pltpu.async_copy(wba_hbm.at[idxB.at[c]], gB.at[c], sem))
    for cp in copies:
        cp.wait()
    accA = jnp.zeros((16,), jnp.float32)
    accB = jnp.zeros((16,), jnp.float32)
    for j in range(NV):
        r, s = j // 8, pl.ds(16 * (j % 8), 16)
        accA = accA + gA[r, s]
        accB = accB + gB[r, s]
    accA_v[:] = accA
    accB_v[:] = accB
    pltpu.sync_copy(accA_v, outA_hbm.at[wid])
    pltpu.sync_copy(accB_v, outB_hbm.at[wid])


_sc_call = pl.kernel(
    _sc_body,
    out_type=(
        jax.ShapeDtypeStruct((NW, 16), jnp.float32),
        jax.ShapeDtypeStruct((NW, 16), jnp.float32),
    ),
    mesh=plsc.VectorSubcoreMesh(core_axis_name="c", subcore_axis_name="s"),
    scratch_types=(
        pltpu.VMEM((BPW,), jnp.int32),
        pltpu.VMEM((BPW,), jnp.int32),
        pltpu.VMEM((NCH, CHUNK), jnp.int32),
        pltpu.VMEM((NCH, CHUNK), jnp.int32),
        pltpu.VMEM((NCH, CHUNK), jnp.float32),
        pltpu.VMEM((NCH, CHUNK), jnp.float32),
        pltpu.VMEM((16,), jnp.float32),
        pltpu.VMEM((16,), jnp.float32),
        pltpu.SemaphoreType.DMA,
    ),
)


def _count_dot(v, tpad):
    q = jnp.right_shift(v, 5)
    s = jnp.bitwise_and(v, 31)
    lvl = lax.broadcasted_iota(jnp.int32, (32, B), 0)
    oh_q = (q[None, :] == lvl).astype(jnp.float32)
    oh_s = (s[None, :] == lvl).astype(jnp.float32)
    cnt = lax.dot_general(oh_q, oh_s, (((1,), (1,)), ((), ())),
                          preferred_element_type=jnp.float32)
    acc = jnp.zeros((32,), jnp.float32)
    for qq in range(32):
        acc = acc + cnt[qq, :] * tpad[32 * qq:32 * qq + 32]
    return jnp.sum(acc)


def _combine_body(w_ref, a_ref, b_ref, tA_ref, tB_ref, pA_ref, pB_ref,
                  out_ref):
    zpad = jnp.zeros((24,), jnp.float32)
    tpadA = jnp.concatenate([tA_ref[:], zpad])
    tpadB = jnp.concatenate([tB_ref[:], zpad])
    S_AB = _count_dot(a_ref[:], tpadA) + jnp.sum(pA_ref[:])
    S_BA = _count_dot(b_ref[:], tpadB) + jnp.sum(pB_ref[:])
    wv = w_ref[:, :]
    la = -jnp.log(1.0 + jnp.exp(-wv))
    l1a = -jnp.log(1.0 + jnp.exp(wv))
    x = la + S_AB
    y = l1a + S_BA
    m = jnp.maximum(x, y)
    out_ref[:, :] = m + jnp.log(jnp.exp(x - m) + jnp.exp(y - m))


_combine_call = pl.pallas_call(
    _combine_body,
    out_shape=jax.ShapeDtypeStruct((1, 1), jnp.float32),
)


def kernel(inputs, w, w_mA, w_cAB, w_mB, w_cBA):
    a = inputs[:, 0]
    b = inputs[:, 1]
    tA, tB = _tc_call(w_mA, w_cAB, w_mB, w_cBA)
    outA, outB = _sc_call(a, b, w_cAB.reshape(-1), w_cBA.reshape(-1))
    res = _combine_call(jnp.reshape(w, (1, 1)), a, b, tA, tB, outA, outB)
    return jnp.reshape(res, ())
